# Initial kernel scaffold; baseline (speedup 1.0000x reference)
import jax
import jax.numpy as jnp
from jax.experimental import pallas as pl


def kernel(bbox_preds, cls_logits):
    raise NotImplementedError("write your pallas kernel here")


# speedup vs baseline: 42.4973x; 42.4973x over previous
import functools

import jax
import jax.numpy as jnp
from jax import lax
from jax.experimental import pallas as pl

N = 20000
NPAD = 20480
R = 160
C = 128
NCLS = 80
PRE_TOP_K = 1000
KEEP_TOP_K = 100
IOU_THRESHOLD = 0.65
SCORE_THRESHOLD = 0.25


def _prep_kernel(lt_ref, bp_ref, sc_ref, lb_ref,
                 x1_ref, y1_ref, x2_ref, y2_ref,
                 ox1_ref, oy1_ref, ox2_ref, oy2_ref,
                 areav_ref, act_ref, fil_ref):
    m0 = lt_ref[pl.ds(0, R), :]
    lbl0 = jnp.zeros((R, C), jnp.int32)

    def cls_body(c, carry):
        m, lbl = carry
        row = lt_ref[pl.ds(c * R, R), :]
        gt = row > m
        return jnp.where(gt, row, m), jnp.where(gt, c, lbl)

    m, lbl = lax.fori_loop(1, NCLS, cls_body, (m0, lbl0))

    rr = lax.broadcasted_iota(jnp.int32, (R, C), 0)
    cc = lax.broadcasted_iota(jnp.int32, (R, C), 1)
    lin = rr * C + cc
    real = lin < N

    scores = 1.0 / (1.0 + jnp.exp(-m))
    scores = jnp.where(real, scores, -1.0)

    cx = bp_ref[pl.ds(0, R), :] * 640.0
    cy = bp_ref[pl.ds(R, R), :] * 640.0
    w = bp_ref[pl.ds(2 * R, R), :] * 100.0 + 1.0
    h = bp_ref[pl.ds(3 * R, R), :] * 100.0 + 1.0
    x1 = cx - w * 0.5
    y1 = cy - h * 0.5
    x2 = cx + w * 0.5
    y2 = cy + h * 0.5
    off = lbl.astype(jnp.float32) * 4096.0
    ox1 = x1 + off
    oy1 = y1 + off
    ox2 = x2 + off
    oy2 = y2 + off

    sbits = lax.bitcast_convert_type(scores, jnp.int32)

    def bis_body(_, lohi):
        lo, hi = lohi
        mid = lo + (hi - lo) // 2
        cnt = jnp.sum((sbits >= mid).astype(jnp.int32))
        ge = cnt >= PRE_TOP_K
        return jnp.where(ge, mid, lo), jnp.where(ge, hi, mid)

    lo0 = jnp.int32(0)
    hi0 = jnp.int32(0x3F800001)
    lo, _ = lax.fori_loop(0, 31, bis_body, (lo0, hi0))

    sel = sbits >= lo
    act = sel & (scores > SCORE_THRESHOLD)

    sc_ref[...] = scores
    lb_ref[...] = lbl
    x1_ref[...] = x1
    y1_ref[...] = y1
    x2_ref[...] = x2
    y2_ref[...] = y2
    ox1_ref[...] = ox1
    oy1_ref[...] = oy1
    ox2_ref[...] = ox2
    oy2_ref[...] = oy2
    areav_ref[...] = (ox2 - ox1) * (oy2 - oy1)
    act_ref[...] = act.astype(jnp.int32)
    fil_ref[...] = sel.astype(jnp.int32)


def _nms_kernel(sc_ref, lb_ref, x1_ref, y1_ref, x2_ref, y2_ref,
                ox1_ref, oy1_ref, ox2_ref, oy2_ref, areav_ref,
                act_ref, fil_ref,
                ob_x1, ob_y1, ob_x2, ob_y2, ob_sc, ob_lb):
    scores = sc_ref[...]
    act0 = act_ref[...]
    fil0 = fil_ref[...]
    rr = lax.broadcasted_iota(jnp.int32, (R, C), 0)
    cc = lax.broadcasted_iota(jnp.int32, (R, C), 1)
    lin = rr * C + cc
    key_act = scores + 2.0
    lin_out = (lax.broadcasted_iota(jnp.int32, (1, C), 0) * C
               + lax.broadcasted_iota(jnp.int32, (1, C), 1))

    lane = lax.broadcasted_iota(jnp.int32, (1, C), 1)

    def ext_f(ref, r, lmask):
        row = ref[pl.ds(r, 1), :]
        return jnp.sum(jnp.where(lmask, row, 0.0))

    def body(i, carry):
        act, fil, o_x1, o_y1, o_x2, o_y2, o_sc, o_lb = carry
        key = jnp.where(act != 0, key_act, jnp.where(fil != 0, scores, -3.0))
        m = jnp.max(key)
        p = jnp.min(jnp.where(key == m, lin, jnp.int32(0x7FFFFFFF)))
        is_kept = m > 2.0
        r = p >> 7
        ln = p & 127
        lmask = lane == ln
        px1 = ext_f(x1_ref, r, lmask)
        py1 = ext_f(y1_ref, r, lmask)
        px2 = ext_f(x2_ref, r, lmask)
        py2 = ext_f(y2_ref, r, lmask)
        pox1 = ext_f(ox1_ref, r, lmask)
        poy1 = ext_f(oy1_ref, r, lmask)
        pox2 = ext_f(ox2_ref, r, lmask)
        poy2 = ext_f(oy2_ref, r, lmask)
        psc = ext_f(sc_ref, r, lmask)
        plb = jnp.sum(jnp.where(lmask, lb_ref[pl.ds(r, 1), :], 0))

        iw = jnp.maximum(
            jnp.minimum(pox2, ox2_ref[...]) - jnp.maximum(pox1, ox1_ref[...]),
            0.0)
        ih = jnp.maximum(
            jnp.minimum(poy2, oy2_ref[...]) - jnp.maximum(poy1, oy1_ref[...]),
            0.0)
        inter = iw * ih
        parea = (pox2 - pox1) * (poy2 - poy1)
        union = parea + areav_ref[...] - inter
        supp = inter > IOU_THRESHOLD * jnp.maximum(union, 1e-6)
        act = jnp.where((supp & is_kept) | (lin == p), 0, act)
        fil = jnp.where(lin == p, 0, fil)

        slot = lin_out == i
        o_x1 = o_x1 + jnp.where(slot, px1, 0.0)
        o_y1 = o_y1 + jnp.where(slot, py1, 0.0)
        o_x2 = o_x2 + jnp.where(slot, px2, 0.0)
        o_y2 = o_y2 + jnp.where(slot, py2, 0.0)
        o_sc = o_sc + jnp.where(slot, jnp.where(is_kept, psc, -1.0), 0.0)
        o_lb = o_lb + jnp.where(slot, jnp.where(is_kept, plb, -1), 0)
        return act, fil, o_x1, o_y1, o_x2, o_y2, o_sc, o_lb

    zf = jnp.zeros((1, C), jnp.float32)
    zi = jnp.zeros((1, C), jnp.int32)
    carry = lax.fori_loop(0, KEEP_TOP_K, body,
                          (act0, fil0, zf, zf, zf, zf, zf, zi))
    _, _, o_x1, o_y1, o_x2, o_y2, o_sc, o_lb = carry
    ob_x1[...] = o_x1
    ob_y1[...] = o_y1
    ob_x2[...] = o_x2
    ob_y2[...] = o_y2
    ob_sc[...] = o_sc
    ob_lb[...] = o_lb


def _f32(shape):
    return jax.ShapeDtypeStruct(shape, jnp.float32)


def _i32(shape):
    return jax.ShapeDtypeStruct(shape, jnp.int32)


@jax.jit
def kernel(bbox_preds, cls_logits):
    lt = cls_logits[0].T
    lt = jnp.pad(lt, ((0, 0), (0, NPAD - N))).reshape(NCLS * R, C)
    bp = bbox_preds[0].T
    bp = jnp.pad(bp, ((0, 0), (0, NPAD - N))).reshape(4 * R, C)

    prep_out = pl.pallas_call(
        _prep_kernel,
        out_shape=[_f32((R, C)), _i32((R, C))]
        + [_f32((R, C))] * 9
        + [_i32((R, C)), _i32((R, C))],
    )(lt, bp)

    nms_out = pl.pallas_call(
        _nms_kernel,
        out_shape=[_f32((1, C))] * 5 + [_i32((1, C))],
    )(*prep_out)

    o_x1, o_y1, o_x2, o_y2, o_sc, o_lb = nms_out
    k = KEEP_TOP_K
    dets = jnp.stack([o_x1[0, :k], o_y1[0, :k], o_x2[0, :k], o_y2[0, :k],
                      o_sc[0, :k]], axis=-1)
    return dets, o_lb[0, :k]
